# baseline (speedup 1.0000x reference)
import functools

import jax
import jax.numpy as jnp
from jax import lax
from jax.experimental import pallas as pl
from jax.experimental.pallas import tpu as pltpu
from jax.experimental.pallas import tpu_sc as plsc

NC, NS = 2, 16
NW = NC * NS
NB = 4
L = 16
HF = 13


def _afm_sc(xi, xw, emb2, emb1, wt, w0b, B, F, D):
    GL = NB * F
    BW = B // NW
    nblk = BW // NB
    mesh = plsc.VectorSubcoreMesh(core_axis_name="c", subcore_axis_name="s")

    @functools.partial(
        pl.kernel,
        out_type=jax.ShapeDtypeStruct((B,), jnp.float32),
        mesh=mesh,
        compiler_params=pltpu.CompilerParams(
            needs_layout_passes=False, use_tc_tiling_on_sc=False),
        scratch_types=[
            pltpu.VMEM((2, GL), jnp.int32),
            pltpu.VMEM((F, BW), jnp.int32),
            pltpu.VMEM((F, BW), jnp.float32),
            pltpu.VMEM((HF, GL, D), jnp.bfloat16),
            pltpu.VMEM((HF, GL, D), jnp.bfloat16),
            pltpu.VMEM((2 * NB, L), jnp.float32),
            pltpu.VMEM((BW,), jnp.float32),
            pltpu.VMEM((D,), jnp.float32),
            pltpu.VMEM((2, L), jnp.float32),
            pltpu.SemaphoreType.DMA,
            pltpu.SemaphoreType.DMA,
            pltpu.SemaphoreType.DMA,
        ],
    )
    def k(xi_hbm, xw_hbm, t2_hbm, t1_hbm, wt_hbm, w0b_hbm, out_hbm,
          xi_v, xd_v, e1w_v, rowsA, rowsB, acc_v, out_v, wt_v, w0b_v,
          semA, semB, semI):
        wid = lax.axis_index("s") * NC + lax.axis_index("c")
        pltpu.sync_copy(wt_hbm, wt_v)
        pltpu.sync_copy(w0b_hbm, w0b_v)
        pltpu.sync_copy(xw_hbm.at[wid], xd_v)
        e1_handles = []
        for i in range(F):
            e1_handles.append(
                pltpu.async_copy(t1_hbm.at[i].at[xd_v.at[i]], e1w_v.at[i], semB))
        for h in e1_handles:
            h.wait()

        wt0 = wt_v[pl.ds(0, L)]
        wt1 = wt_v[pl.ds(L, L)]
        w0v = w0b_v[0]
        bias = w0b_v[1]
        lane0 = lax.iota(jnp.int32, L) == 0
        zero = jnp.zeros((L,), jnp.float32)

        def comp_half(rows, i0):
            def comp(ii, c):
                i = ii + i0
                for bb in range(NB):
                    base = bb * F
                    s0, s1 = plsc.unpack(
                        rows[ii, base, :], format=plsc.PackFormat.INTERLEAVED)
                    for j in range(1, F):
                        a, b = plsc.unpack(
                            rows[ii, base + j, :],
                            format=plsc.PackFormat.INTERLEAVED)
                        s0 = s0 + a
                        s1 = s1 + b
                    d0, d1 = plsc.unpack(
                        rows[ii, base + i, :], format=plsc.PackFormat.INTERLEAVED)
                    plsc.addupdate(acc_v.at[2 * bb], d0 * s0)
                    plsc.addupdate(acc_v.at[2 * bb + 1], d1 * s1)
                return c
            lax.fori_loop(0, HF, comp, 0)

        pltpu.sync_copy(xi_hbm.at[wid * nblk], xi_v.at[0])
        for i in range(HF):
            pltpu.async_copy(t2_hbm.at[i].at[xi_v.at[0]], rowsA.at[i], semA)

        def block(k_, carry):
            p = lax.rem(k_, 2)
            pn = 1 - p
            hB = []
            for i in range(HF, F):
                hB.append(pltpu.async_copy(
                    t2_hbm.at[i].at[xi_v.at[p]], rowsB.at[i - HF], semB))
            kc = jnp.minimum(k_ + 1, nblk - 1)
            hI = pltpu.async_copy(xi_hbm.at[wid * nblk + kc], xi_v.at[pn], semI)
            for r in range(2 * NB):
                acc_v[r] = zero
            for i in range(HF):
                pltpu.make_async_copy(
                    t2_hbm.at[0].at[pl.ds(0, GL)], rowsA.at[i], semA).wait()
            comp_half(rowsA, 0)
            hI.wait()
            for i in range(HF):
                pltpu.async_copy(
                    t2_hbm.at[i].at[xi_v.at[pn]], rowsA.at[i], semA)
            for h in hB:
                h.wait()
            comp_half(rowsB, HF)

            for bb in range(NB):
                b_local = k_ * NB + bb
                a0 = acc_v[2 * bb]
                a1 = acc_v[2 * bb + 1]
                tvec = a0 * wt0 + a1 * wt1 + bias
                tot = jnp.broadcast_to(jnp.sum(tvec), (L,))
                pos = jnp.broadcast_to(b_local, (L,))
                plsc.store_scatter(out_v, [pos], tot, mask=lane0)
            return carry

        lax.fori_loop(0, nblk, block, 0)
        for i in range(HF):
            pltpu.make_async_copy(
                t2_hbm.at[0].at[pl.ds(0, GL)], rowsA.at[i], semA).wait()

        for g in range(BW // L):
            fvec = e1w_v[0, pl.ds(g * L, L)]
            for i in range(1, F):
                fvec = fvec + e1w_v[i, pl.ds(g * L, L)]
            out_v[pl.ds(g * L, L)] = out_v[pl.ds(g * L, L)] + w0v * fvec
        pltpu.sync_copy(out_v, out_hbm.at[pl.ds(wid * BW, BW)])

    return k(xi, xw, emb2, emb1, wt, w0b)


def kernel(x, emb1, emb2, attention_W, out_W, out_b):
    B = x.shape[0]
    F, V, D = emb2.shape
    x32 = x.astype(jnp.int32)
    xi = x32.reshape(B // NB, NB * F)
    xw = x32.reshape(NW, B // NW, F).transpose(0, 2, 1)
    t1 = emb1.reshape(F, V)
    t2 = emb2.astype(jnp.bfloat16)
    wt = out_W[1:, 0].astype(jnp.float32)
    wt = jnp.concatenate([wt[0::2], wt[1::2]])
    w0b = jnp.stack([
        jnp.full((L,), out_W[0, 0], jnp.float32),
        jnp.full((L,), out_b[0] / L, jnp.float32),
    ])
    out = _afm_sc(xi, xw, t2, t1, wt, w0b, B, F, D)
    return out.reshape(B, 1)


# speedup vs baseline: 1.1913x; 1.1913x over previous
import functools

import jax
import jax.numpy as jnp
from jax import lax
from jax.experimental import pallas as pl
from jax.experimental.pallas import tpu as pltpu
from jax.experimental.pallas import tpu_sc as plsc

NC, NS = 2, 16
NW = NC * NS
NB = 4
L = 16
HF = 13


def _afm_sc(xi, xw, emb2, emb1, wt, w0b, B, F, D):
    GL = NB * F
    BW = B // NW
    nblk = BW // NB
    mesh = plsc.VectorSubcoreMesh(core_axis_name="c", subcore_axis_name="s")

    @functools.partial(
        pl.kernel,
        out_type=jax.ShapeDtypeStruct((B,), jnp.float32),
        mesh=mesh,
        compiler_params=pltpu.CompilerParams(
            needs_layout_passes=False, use_tc_tiling_on_sc=False),
        scratch_types=[
            pltpu.VMEM((2, GL), jnp.int32),
            pltpu.VMEM((F, BW), jnp.int32),
            pltpu.VMEM((F, BW), jnp.float32),
            pltpu.VMEM((HF, GL, D), jnp.float32),
            pltpu.VMEM((HF, GL, D), jnp.float32),
            pltpu.VMEM((2 * NB, L), jnp.float32),
            pltpu.VMEM((BW,), jnp.float32),
            pltpu.VMEM((D,), jnp.float32),
            pltpu.VMEM((2, L), jnp.float32),
            pltpu.SemaphoreType.DMA,
            pltpu.SemaphoreType.DMA,
            pltpu.SemaphoreType.DMA,
        ],
    )
    def k(xi_hbm, xw_hbm, t2_hbm, t1_hbm, wt_hbm, w0b_hbm, out_hbm,
          xi_v, xd_v, e1w_v, rowsA, rowsB, acc_v, out_v, wt_v, w0b_v,
          semA, semB, semI):
        wid = lax.axis_index("s") * NC + lax.axis_index("c")
        pltpu.sync_copy(wt_hbm, wt_v)
        pltpu.sync_copy(w0b_hbm, w0b_v)
        pltpu.sync_copy(xw_hbm.at[wid], xd_v)
        e1_handles = []
        for i in range(F):
            e1_handles.append(
                pltpu.async_copy(t1_hbm.at[i].at[xd_v.at[i]], e1w_v.at[i], semB))
        for h in e1_handles:
            h.wait()

        wt0 = wt_v[pl.ds(0, L)]
        wt1 = wt_v[pl.ds(L, L)]
        w0v = w0b_v[0]
        bias = w0b_v[1]
        lane0 = lax.iota(jnp.int32, L) == 0
        zero = jnp.zeros((L,), jnp.float32)

        def comp_half(rows, i0):
            def comp(ii, c):
                i = ii + i0
                for bb in range(NB):
                    base = bb * F
                    s0 = rows[ii, base, pl.ds(0, L)]
                    s1 = rows[ii, base, pl.ds(L, L)]
                    for j in range(1, F):
                        s0 = s0 + rows[ii, base + j, pl.ds(0, L)]
                        s1 = s1 + rows[ii, base + j, pl.ds(L, L)]
                    d0 = rows[ii, base + i, pl.ds(0, L)]
                    d1 = rows[ii, base + i, pl.ds(L, L)]
                    plsc.addupdate(acc_v.at[2 * bb], d0 * s0)
                    plsc.addupdate(acc_v.at[2 * bb + 1], d1 * s1)
                return c
            lax.fori_loop(0, HF, comp, 0)

        pltpu.sync_copy(xi_hbm.at[wid * nblk], xi_v.at[0])
        for i in range(HF):
            pltpu.async_copy(t2_hbm.at[i].at[xi_v.at[0]], rowsA.at[i], semA)

        def block(k_, carry):
            p = lax.rem(k_, 2)
            pn = 1 - p
            hB = []
            for i in range(HF, F):
                hB.append(pltpu.async_copy(
                    t2_hbm.at[i].at[xi_v.at[p]], rowsB.at[i - HF], semB))
            kc = jnp.minimum(k_ + 1, nblk - 1)
            hI = pltpu.async_copy(xi_hbm.at[wid * nblk + kc], xi_v.at[pn], semI)
            for r in range(2 * NB):
                acc_v[r] = zero
            for i in range(HF):
                pltpu.make_async_copy(
                    t2_hbm.at[0].at[pl.ds(0, GL)], rowsA.at[i], semA).wait()
            comp_half(rowsA, 0)
            hI.wait()
            for i in range(HF):
                pltpu.async_copy(
                    t2_hbm.at[i].at[xi_v.at[pn]], rowsA.at[i], semA)
            for h in hB:
                h.wait()
            comp_half(rowsB, HF)

            for bb in range(NB):
                b_local = k_ * NB + bb
                a0 = acc_v[2 * bb]
                a1 = acc_v[2 * bb + 1]
                tvec = a0 * wt0 + a1 * wt1 + bias
                tot = jnp.broadcast_to(jnp.sum(tvec), (L,))
                pos = jnp.broadcast_to(b_local, (L,))
                plsc.store_scatter(out_v, [pos], tot, mask=lane0)
            return carry

        lax.fori_loop(0, nblk, block, 0)
        for i in range(HF):
            pltpu.make_async_copy(
                t2_hbm.at[0].at[pl.ds(0, GL)], rowsA.at[i], semA).wait()

        for g in range(BW // L):
            fvec = e1w_v[0, pl.ds(g * L, L)]
            for i in range(1, F):
                fvec = fvec + e1w_v[i, pl.ds(g * L, L)]
            out_v[pl.ds(g * L, L)] = out_v[pl.ds(g * L, L)] + w0v * fvec
        pltpu.sync_copy(out_v, out_hbm.at[pl.ds(wid * BW, BW)])

    return k(xi, xw, emb2, emb1, wt, w0b)


def kernel(x, emb1, emb2, attention_W, out_W, out_b):
    B = x.shape[0]
    F, V, D = emb2.shape
    x32 = x.astype(jnp.int32)
    xi = x32.reshape(B // NB, NB * F)
    xw = x32.reshape(NW, B // NW, F).transpose(0, 2, 1)
    t1 = emb1.reshape(F, V)
    wt = out_W[1:, 0].astype(jnp.float32)
    w0b = jnp.stack([
        jnp.full((L,), out_W[0, 0], jnp.float32),
        jnp.full((L,), out_b[0] / L, jnp.float32),
    ])
    out = _afm_sc(xi, xw, emb2, t1, wt, w0b, B, F, D)
    return out.reshape(B, 1)
